# Initial kernel scaffold; baseline (speedup 1.0000x reference)
import jax
import jax.numpy as jnp
from jax.experimental import pallas as pl


def kernel(input_ids, embed_table, W, b):
    raise NotImplementedError("write your pallas kernel here")


# speedup vs baseline: 2.8082x; 2.8082x over previous
import functools

import jax
import jax.numpy as jnp
from jax import lax
from jax.experimental import pallas as pl
from jax.experimental.pallas import tpu as pltpu
from jax.experimental.pallas import tpu_sc as plsc

VOCAB = 16
D = 16
CHUNK = 2048


def _h_body(e_ref, w_ref, b_ref, h_ref):
    h = lax.dot_general(
        e_ref[...], w_ref[...],
        (((1,), (1,)), ((), ())),
        preferred_element_type=jnp.float32,
    )
    h_ref[...] = h + b_ref[...]


def _compute_h(embed_table, W, b):
    b_mat = jnp.broadcast_to(b.reshape(1, D), (VOCAB, D))
    return pl.pallas_call(
        _h_body,
        out_shape=jax.ShapeDtypeStruct((VOCAB, D), jnp.float32),
    )(embed_table, W, b_mat)


def _sc_gather(ids, h_flat):
    n = ids.shape[0]
    info = plsc.get_sparse_core_info()
    nc, ns = info.num_cores, info.num_subcores
    nw = nc * ns
    per_w = n // nw
    assert per_w * nw == n and per_w % CHUNK == 0
    n_chunks = per_w // CHUNK

    mesh = plsc.VectorSubcoreMesh(core_axis_name="c", subcore_axis_name="s")

    @functools.partial(
        pl.kernel,
        out_type=[
            jax.ShapeDtypeStruct((n * D,), jnp.float32),
            jax.ShapeDtypeStruct((n * D,), jnp.float32),
        ],
        mesh=mesh,
        scratch_types=[
            pltpu.VMEM((VOCAB * D,), jnp.float32),
            pltpu.VMEM((CHUNK,), jnp.int32),
            pltpu.VMEM((CHUNK * D,), jnp.float32),
            pltpu.VMEM((CHUNK * D,), jnp.float32),
        ],
        compiler_params=pltpu.CompilerParams(needs_layout_passes=False),
    )
    def k(ids_hbm, h_hbm, hid_hbm, log_hbm, h_v, ids_v, hid_v, log_v):
        wid = lax.axis_index("s") * nc + lax.axis_index("c")
        base = wid * per_w
        pltpu.sync_copy(h_hbm, h_v)
        lane16 = lax.iota(jnp.int32, 16) * D

        def chunk_body(c, carry):
            off = base + c * CHUNK
            pltpu.sync_copy(ids_hbm.at[pl.ds(off, CHUNK)], ids_v)

            def group_body(g, carry2):
                idv = ids_v[pl.ds(g * 16, 16)]
                bi = idv * D
                g0 = plsc.load_gather(h_v, (bi,))
                pos0 = lane16 + g * (16 * D)
                for j in range(D):
                    r = plsc.load_gather(h_v, (bi + j,))
                    plsc.store_scatter(hid_v, (pos0 + j,), r)
                    plsc.store_scatter(log_v, (pos0 + j,), g0)
                return carry2

            lax.fori_loop(0, CHUNK // 16, group_body, 0, unroll=False)
            pltpu.sync_copy(hid_v, hid_hbm.at[pl.ds(off * D, CHUNK * D)])
            pltpu.sync_copy(log_v, log_hbm.at[pl.ds(off * D, CHUNK * D)])
            return carry

        lax.fori_loop(0, n_chunks, chunk_body, 0, unroll=False)

    return k(ids, h_flat)


def kernel(input_ids, embed_table, W, b):
    bsz, seq = input_ids.shape
    ids = input_ids.reshape(-1).astype(jnp.int32)
    h = _compute_h(embed_table, W, b)
    hid_flat, log_flat = _sc_gather(ids, h.reshape(-1))
    hidden = hid_flat.reshape(bsz, seq, D)
    logits = log_flat.reshape(bsz, seq, D)
    return (logits, hidden)
